# speedup vs baseline: 1.6513x; 1.6513x over previous
import functools

import jax
import jax.numpy as jnp
from jax import lax
from jax.experimental import pallas as pl
from jax.experimental.pallas import tpu as pltpu
from jax.experimental.pallas import tpu_sc as plsc

B, S, NUM_VQ = 4, 8192, 4
D = 1024
T = B * S
NUM_AUDIO = 8192
NC, NS = 2, 16
NW = NC * NS
CHUNK = T // NW
NB = 16
NBLK = CHUNK // NB
LANES = 16


def _make_kernel():
    mesh = plsc.VectorSubcoreMesh(core_axis_name="c", subcore_axis_name="s")

    @functools.partial(
        pl.kernel,
        out_type=jax.ShapeDtypeStruct((T, D), jnp.float32),
        mesh=mesh,
        scratch_types=[
            pltpu.VMEM((NUM_VQ * CHUNK,), jnp.int32),
            pltpu.VMEM((CHUNK,), jnp.int32),
            pltpu.VMEM((NUM_VQ * NB, D), jnp.float32),
            pltpu.VMEM((NB, D), jnp.float32),
            pltpu.VMEM((NB, D), jnp.float32),
            pltpu.VMEM((NUM_VQ * NB,), jnp.int32),
            pltpu.SemaphoreType.DMA,
            pltpu.SemaphoreType.DMA,
        ],
    )
    def body(ids_hbm, mask_hbm, text_hbm, code_hbm, out_hbm,
             ids_v, mask_v, gbuf, acc, tbuf, cidx, sem_c, sem_t):
        wid = lax.axis_index("s") * NC + lax.axis_index("c")
        base = wid * CHUNK
        for i in range(NUM_VQ):
            pltpu.sync_copy(ids_hbm.at[pl.ds(i * T + base, CHUNK)],
                            ids_v.at[pl.ds(i * CHUNK, CHUNK)])
        pltpu.sync_copy(mask_hbm.at[pl.ds(base, CHUNK)], mask_v)

        def blk(b, carry):
            t0 = b * NB
            for i in range(NUM_VQ):
                idv = ids_v[pl.ds(i * CHUNK + t0, NB)]
                cidx[pl.ds(i * NB, NB)] = idv + i * NUM_AUDIO
            gcopy = pltpu.async_copy(code_hbm.at[cidx], gbuf, sem_c)
            tcopy = pltpu.async_copy(text_hbm.at[ids_v.at[pl.ds(t0, NB)]],
                                     tbuf, sem_t)
            gcopy.wait()

            def addv(c, _):
                off = c * LANES
                for k in range(NB):
                    acc[k, pl.ds(off, LANES)] = (
                        gbuf[k, pl.ds(off, LANES)]
                        + gbuf[NB + k, pl.ds(off, LANES)]
                        + gbuf[2 * NB + k, pl.ds(off, LANES)]
                        + gbuf[3 * NB + k, pl.ds(off, LANES)]
                    )
                return 0
            lax.fori_loop(0, D // LANES, addv, 0, unroll=False)

            tcopy.wait()
            mv = mask_v[pl.ds(t0, NB)]
            for k in range(NB):
                @pl.when(mv[k] != 0)
                def _copy_text(k=k):
                    def cp(c, _):
                        acc[k, pl.ds(c * LANES, LANES)] = \
                            tbuf[k, pl.ds(c * LANES, LANES)]
                        return 0
                    lax.fori_loop(0, D // LANES, cp, 0, unroll=False)

            pltpu.sync_copy(acc, out_hbm.at[pl.ds(base + t0, NB)])
            return carry

        lax.fori_loop(0, NBLK, blk, 0, unroll=False)

    return body


def kernel(input_ids, text_mask, text_table, code_tables):
    ids_t = input_ids.reshape(T, NUM_VQ).T.reshape(-1)
    maski = text_mask.reshape(T).astype(jnp.int32)
    code_flat = code_tables.reshape(NUM_VQ * NUM_AUDIO, D)
    out = _make_kernel()(ids_t, maski, text_table, code_flat)
    return out.reshape(B, S, D)


# speedup vs baseline: 3.3239x; 2.0129x over previous
import functools

import jax
import jax.numpy as jnp
from jax import lax
from jax.experimental import pallas as pl
from jax.experimental.pallas import tpu as pltpu
from jax.experimental.pallas import tpu_sc as plsc

B, S, NUM_VQ = 4, 8192, 4
D = 1024
T = B * S
NUM_AUDIO = 8192
NC, NS = 2, 16
NW = NC * NS
CHUNK = T // NW
LANES = 16
NB = 16
GROWS = NUM_VQ * NB


def _make_kernel():
    mesh = plsc.VectorSubcoreMesh(core_axis_name="c", subcore_axis_name="s")

    @functools.partial(
        pl.kernel,
        out_type=jax.ShapeDtypeStruct((T, D), jnp.float32),
        mesh=mesh,
        scratch_types=[
            pltpu.VMEM((NUM_VQ * CHUNK,), jnp.int32),
            pltpu.VMEM((CHUNK,), jnp.int32),
            pltpu.VMEM((CHUNK,), jnp.int32),
            pltpu.VMEM((CHUNK,), jnp.int32),
            pltpu.VMEM((CHUNK,), jnp.int32),
            pltpu.VMEM((NUM_VQ * CHUNK,), jnp.int32),
            pltpu.VMEM((GROWS, D), jnp.float32),
            pltpu.VMEM((NB, D), jnp.float32),
            pltpu.VMEM((NB, D), jnp.float32),
            pltpu.SemaphoreType.DMA,
            pltpu.SemaphoreType.DMA,
        ],
    )
    def body(ids_hbm, mask_hbm, text_hbm, code_hbm, out_hbm,
             ids_v, mask_v, tpos, tids, cpos, cids, gbuf, acc, tbuf,
             sem_g, sem_s):
        wid = lax.axis_index("s") * NC + lax.axis_index("c")
        base = wid * CHUNK
        for i in range(NUM_VQ):
            pltpu.sync_copy(ids_hbm.at[pl.ds(i * T + base, CHUNK)],
                            ids_v.at[pl.ds(i * CHUNK, CHUNK)])
        pltpu.sync_copy(mask_hbm.at[pl.ds(base, CHUNK)], mask_v)

        iota = lax.iota(jnp.int32, LANES)

        def lane_gather(v, idx):
            return lax.gather(
                v, idx[:, None],
                lax.GatherDimensionNumbers(
                    offset_dims=(), collapsed_slice_dims=(0,),
                    start_index_map=(0,)),
                (1,), mode=lax.GatherScatterMode.PROMISE_IN_BOUNDS)

        def prefix_sum(v):
            for d in (1, 2, 4, 8):
                sh = lane_gather(v, jnp.maximum(iota - d, 0))
                v = v + jnp.where(iota >= d, sh, 0)
            return v

        def lower_bound(arr):
            tgt = iota + 1
            lo = jnp.zeros((LANES,), jnp.int32)
            for step in (8, 4, 2, 1):
                v = lane_gather(arr, lo + (step - 1))
                lo = jnp.where(v < tgt, lo + step, lo)
            return lo

        def part(j, carry):
            (bt, ft, cr_tp, cr_ti,
             bc, fc, cr_cp, cr_c0, cr_c1, cr_c2, cr_c3) = carry
            o = j * LANES
            mi = mask_v[pl.ds(o, LANES)]
            pos = base + o + iota
            pt = prefix_sum(mi)
            pc = (1 + iota) - pt
            cnt_t = pt[LANES - 1]
            cnt_c = LANES - cnt_t

            def emit(arr_prefix, cnt, fill, blk, vals, carries, stores):
                src = lower_bound(arr_prefix)
                tot = fill + cnt
                full = (tot >= LANES).astype(jnp.int32)
                new_carries = []
                merged_all = []
                for x, cr in zip(vals, carries):
                    cx = lane_gather(x, src)
                    merged = jnp.where(iota < fill,
                                       cr,
                                       lane_gather(cx, jnp.maximum(iota - fill, 0)))
                    spill = lane_gather(cx, jnp.minimum(iota + LANES - fill,
                                                        LANES - 1))
                    merged_all.append(merged)
                    new_carries.append(jnp.where(full > 0, spill, merged))

                @pl.when(full > 0)
                def _():
                    for st, mv in zip(stores, merged_all):
                        st(blk, mv)
                return blk + full, tot - LANES * full, new_carries

            def t_store_pos(b2, v):
                tpos[pl.ds(b2 * NB, NB)] = v

            def t_store_id(b2, v):
                tids[pl.ds(b2 * NB, NB)] = v

            def c_store_pos(b2, v):
                cpos[pl.ds(b2 * NB, NB)] = v

            def c_store_id(i):
                def st(b2, v):
                    cids[pl.ds(b2 * GROWS + i * NB, NB)] = v
                return st

            id0 = ids_v[pl.ds(o, LANES)]
            bt, ft, (cr_tp, cr_ti) = emit(
                pt, cnt_t, ft, bt, [pos, id0], [cr_tp, cr_ti],
                [t_store_pos, t_store_id])

            cvals = [pos] + [ids_v[pl.ds(i * CHUNK + o, LANES)] + i * NUM_AUDIO
                             for i in range(NUM_VQ)]
            bc, fc, (cr_cp, cr_c0, cr_c1, cr_c2, cr_c3) = emit(
                pc, cnt_c, fc, bc, cvals,
                [cr_cp, cr_c0, cr_c1, cr_c2, cr_c3],
                [c_store_pos] + [c_store_id(i) for i in range(NUM_VQ)])

            return (bt, ft, cr_tp, cr_ti,
                    bc, fc, cr_cp, cr_c0, cr_c1, cr_c2, cr_c3)

        zvec = jnp.zeros((LANES,), jnp.int32)
        z = jnp.int32(0)
        (bt, ft, cr_tp, cr_ti,
         bc, fc, cr_cp, cr_c0, cr_c1, cr_c2, cr_c3) = lax.fori_loop(
            0, CHUNK // LANES, part,
            (z, z, zvec, zvec, z, z, zvec, zvec, zvec, zvec, zvec))

        @pl.when(ft > 0)
        def _flush_text():
            sel = jnp.minimum(iota, ft - 1)
            tpos[pl.ds(bt * NB, NB)] = lane_gather(cr_tp, sel)
            tids[pl.ds(bt * NB, NB)] = lane_gather(cr_ti, sel)

        nbt = bt + (ft > 0).astype(jnp.int32)

        @pl.when(fc > 0)
        def _flush_code():
            sel = jnp.minimum(iota, fc - 1)
            cpos[pl.ds(bc * NB, NB)] = lane_gather(cr_cp, sel)
            for i, cr in enumerate((cr_c0, cr_c1, cr_c2, cr_c3)):
                cids[pl.ds(bc * GROWS + i * NB, NB)] = lane_gather(cr, sel)

        nbc = bc + (fc > 0).astype(jnp.int32)

        def tblk(b, carry):
            pltpu.async_copy(text_hbm.at[tids.at[pl.ds(b * NB, NB)]],
                             tbuf, sem_g).wait()
            tposv = tpos[pl.ds(b * NB, NB)]
            pltpu.async_copy(tbuf, out_hbm.at[tposv], sem_s).wait()
            return carry

        lax.fori_loop(0, nbt, tblk, 0, unroll=False)

        def cblk(b, carry):
            pltpu.async_copy(code_hbm.at[cids.at[pl.ds(b * GROWS, GROWS)]],
                             gbuf, sem_g).wait()

            def addv(c, _):
                off = c * LANES
                for k in range(NB):
                    acc[k, pl.ds(off, LANES)] = (
                        gbuf[k, pl.ds(off, LANES)]
                        + gbuf[NB + k, pl.ds(off, LANES)]
                        + gbuf[2 * NB + k, pl.ds(off, LANES)]
                        + gbuf[3 * NB + k, pl.ds(off, LANES)]
                    )
                return 0
            lax.fori_loop(0, D // LANES, addv, 0, unroll=False)

            cposv = cpos[pl.ds(b * NB, NB)]
            pltpu.async_copy(acc, out_hbm.at[cposv], sem_s).wait()
            return carry

        lax.fori_loop(0, nbc, cblk, 0, unroll=False)

    return body


def kernel(input_ids, text_mask, text_table, code_tables):
    ids_t = input_ids.reshape(T, NUM_VQ).T.reshape(-1)
    maski = text_mask.reshape(T).astype(jnp.int32)
    code_flat = code_tables.reshape(NUM_VQ * NUM_AUDIO, D)
    out = _make_kernel()(ids_t, maski, text_table, code_flat)
    return out.reshape(B, S, D)


# speedup vs baseline: 3.4334x; 1.0329x over previous
import functools

import jax
import jax.numpy as jnp
from jax import lax
from jax.experimental import pallas as pl
from jax.experimental.pallas import tpu as pltpu
from jax.experimental.pallas import tpu_sc as plsc

B, S, NUM_VQ = 4, 8192, 4
D = 1024
T = B * S
NUM_AUDIO = 8192
NC, NS = 2, 16
NW = NC * NS
CHUNK = T // NW
LANES = 16
NB = 16
GROWS = NUM_VQ * NB


def _make_kernel():
    mesh = plsc.VectorSubcoreMesh(core_axis_name="c", subcore_axis_name="s")

    @functools.partial(
        pl.kernel,
        out_type=jax.ShapeDtypeStruct((T, D), jnp.float32),
        mesh=mesh,
        scratch_types=[
            pltpu.VMEM((NUM_VQ * CHUNK,), jnp.int32),
            pltpu.VMEM((CHUNK,), jnp.int32),
            pltpu.VMEM((CHUNK,), jnp.int32),
            pltpu.VMEM((CHUNK,), jnp.int32),
            pltpu.VMEM((CHUNK,), jnp.int32),
            pltpu.VMEM((NUM_VQ * CHUNK,), jnp.int32),
            pltpu.VMEM((GROWS, D), jnp.float32),
            pltpu.VMEM((NB, D), jnp.float32),
            pltpu.VMEM((NB, D), jnp.float32),
            pltpu.SemaphoreType.DMA,
            pltpu.SemaphoreType.DMA,
            pltpu.SemaphoreType.DMA,
            pltpu.SemaphoreType.DMA,
            pltpu.SemaphoreType.DMA,
            pltpu.SemaphoreType.DMA,
        ],
    )
    def body(ids_hbm, mask_hbm, text_hbm, code_hbm, out_hbm,
             ids_v, mask_v, tpos, tids, cpos, cids, gbuf, acc0, acc1,
             sg0, sg1, sg2, sg3, ss0, ss1):
        accs = (acc0, acc1)
        sgs = (sg0, sg1, sg2, sg3)
        sss = (ss0, ss1)

        wid = lax.axis_index("s") * NC + lax.axis_index("c")
        base = wid * CHUNK
        for i in range(NUM_VQ):
            pltpu.sync_copy(ids_hbm.at[pl.ds(i * T + base, CHUNK)],
                            ids_v.at[pl.ds(i * CHUNK, CHUNK)])
        pltpu.sync_copy(mask_hbm.at[pl.ds(base, CHUNK)], mask_v)

        iota = lax.iota(jnp.int32, LANES)

        def lane_gather(v, idx):
            return lax.gather(
                v, idx[:, None],
                lax.GatherDimensionNumbers(
                    offset_dims=(), collapsed_slice_dims=(0,),
                    start_index_map=(0,)),
                (1,), mode=lax.GatherScatterMode.PROMISE_IN_BOUNDS)

        def prefix_sum(v):
            for d in (1, 2, 4, 8):
                sh = lane_gather(v, jnp.maximum(iota - d, 0))
                v = v + jnp.where(iota >= d, sh, 0)
            return v

        def lower_bound(arr):
            tgt = iota + 1
            lo = jnp.zeros((LANES,), jnp.int32)
            for step in (8, 4, 2, 1):
                v = lane_gather(arr, lo + (step - 1))
                lo = jnp.where(v < tgt, lo + step, lo)
            return lo

        def part(j, carry):
            (bt, ft, cr_tp, cr_ti,
             bc, fc, cr_cp, cr_c0, cr_c1, cr_c2, cr_c3) = carry
            o = j * LANES
            mi = mask_v[pl.ds(o, LANES)]
            pos = base + o + iota
            pt = prefix_sum(mi)
            pc = (1 + iota) - pt
            cnt_t = pt[LANES - 1]
            cnt_c = LANES - cnt_t

            def emit(arr_prefix, cnt, fill, blk, vals, carries, stores):
                src = lower_bound(arr_prefix)
                tot = fill + cnt
                full = (tot >= LANES).astype(jnp.int32)
                new_carries = []
                merged_all = []
                for x, cr in zip(vals, carries):
                    cx = lane_gather(x, src)
                    merged = jnp.where(iota < fill,
                                       cr,
                                       lane_gather(cx, jnp.maximum(iota - fill, 0)))
                    spill = lane_gather(cx, jnp.minimum(iota + LANES - fill,
                                                        LANES - 1))
                    merged_all.append(merged)
                    new_carries.append(jnp.where(full > 0, spill, merged))

                @pl.when(full > 0)
                def _():
                    for st, mv in zip(stores, merged_all):
                        st(blk, mv)
                return blk + full, tot - LANES * full, new_carries

            def t_store_pos(b2, v):
                tpos[pl.ds(b2 * NB, NB)] = v

            def t_store_id(b2, v):
                tids[pl.ds(b2 * NB, NB)] = v

            def c_store_pos(b2, v):
                cpos[pl.ds(b2 * NB, NB)] = v

            def c_store_id(i):
                def st(b2, v):
                    cids[pl.ds(b2 * GROWS + i * NB, NB)] = v
                return st

            id0 = ids_v[pl.ds(o, LANES)]
            bt, ft, (cr_tp, cr_ti) = emit(
                pt, cnt_t, ft, bt, [pos, id0], [cr_tp, cr_ti],
                [t_store_pos, t_store_id])

            cvals = [pos] + [ids_v[pl.ds(i * CHUNK + o, LANES)] + i * NUM_AUDIO
                             for i in range(NUM_VQ)]
            bc, fc, (cr_cp, cr_c0, cr_c1, cr_c2, cr_c3) = emit(
                pc, cnt_c, fc, bc, cvals,
                [cr_cp, cr_c0, cr_c1, cr_c2, cr_c3],
                [c_store_pos] + [c_store_id(i) for i in range(NUM_VQ)])

            return (bt, ft, cr_tp, cr_ti,
                    bc, fc, cr_cp, cr_c0, cr_c1, cr_c2, cr_c3)

        zvec = jnp.zeros((LANES,), jnp.int32)
        z = jnp.int32(0)
        (bt, ft, cr_tp, cr_ti,
         bc, fc, cr_cp, cr_c0, cr_c1, cr_c2, cr_c3) = lax.fori_loop(
            0, CHUNK // LANES, part,
            (z, z, zvec, zvec, z, z, zvec, zvec, zvec, zvec, zvec))

        @pl.when(ft > 0)
        def _flush_text():
            sel = jnp.minimum(iota, ft - 1)
            tpos[pl.ds(bt * NB, NB)] = lane_gather(cr_tp, sel)
            tids[pl.ds(bt * NB, NB)] = lane_gather(cr_ti, sel)

        nbt = bt + (ft > 0).astype(jnp.int32)

        @pl.when(fc > 0)
        def _flush_code():
            sel = jnp.minimum(iota, fc - 1)
            cpos[pl.ds(bc * NB, NB)] = lane_gather(cr_cp, sel)
            for i, cr in enumerate((cr_c0, cr_c1, cr_c2, cr_c3)):
                cids[pl.ds(bc * GROWS + i * NB, NB)] = lane_gather(cr, sel)

        nbc = bc + (fc > 0).astype(jnp.int32)

        def t_gather(b, k):
            return pltpu.make_async_copy(
                text_hbm.at[tids.at[pl.ds(b * NB, NB)]], accs[k], sgs[k])

        def t_scatter(b, k):
            tposv = tpos[pl.ds(b * NB, NB)]
            return pltpu.make_async_copy(accs[k], out_hbm.at[tposv], sss[k])

        for k in range(2):
            @pl.when(k < nbt)
            def _(k=k):
                t_gather(k, k).start()

        def tloop(i, carry):
            for k in range(2):
                b = 2 * i + k

                @pl.when(b < nbt)
                def _(b=b, k=k):
                    t_gather(b, k).wait()
                    t_scatter(b, k).start()
            for k in range(2):
                bn = 2 * i + 2 + k

                @pl.when(bn < nbt)
                def _(bn=bn, k=k, i=i):
                    t_scatter(2 * i + k, k).wait()
                    t_gather(bn, k).start()
            return carry

        lax.fori_loop(0, (nbt + 1) >> 1, tloop, 0, unroll=False)

        for k in range(2):
            @pl.when(nbt > k)
            def _(k=k):
                lb = ((nbt - 1 - k) >> 1) * 2 + k
                t_scatter(lb, k).wait()

        def c_gather(b, i):
            return pltpu.make_async_copy(
                code_hbm.at[cids.at[pl.ds(b * GROWS + i * NB, NB)]],
                gbuf.at[pl.ds(i * NB, NB)], sgs[i])

        def c_scatter(b, p):
            cposv = cpos[pl.ds(b * NB, NB)]
            return pltpu.make_async_copy(accs[p], out_hbm.at[cposv], sss[p])

        @pl.when(nbc > 0)
        def _():
            for i in range(NUM_VQ):
                c_gather(0, i).start()

        def cblock(b, p):
            for i in range(NUM_VQ):
                c_gather(b, i).wait()
                if i == 0:
                    @pl.when(b >= 2)
                    def _(b=b, p=p):
                        c_scatter(b - 2, p).wait()

                    def cp0(c, _):
                        off = c * LANES
                        for r in range(NB):
                            accs[p][r, pl.ds(off, LANES)] = \
                                gbuf[r, pl.ds(off, LANES)]
                        return 0
                    lax.fori_loop(0, D // LANES, cp0, 0, unroll=False)
                else:
                    def addp(c, _, i=i):
                        off = c * LANES
                        for r in range(NB):
                            plsc.addupdate(
                                accs[p].at[r, pl.ds(off, LANES)],
                                gbuf[i * NB + r, pl.ds(off, LANES)])
                        return 0
                    lax.fori_loop(0, D // LANES, addp, 0, unroll=False)

                @pl.when(b + 1 < nbc)
                def _(b=b, i=i):
                    c_gather(b + 1, i).start()
            c_scatter(b, p).start()

        def cloop(i2, carry):
            for p in range(2):
                b = 2 * i2 + p

                @pl.when(b < nbc)
                def _(b=b, p=p):
                    cblock(b, p)
            return carry

        lax.fori_loop(0, (nbc + 1) >> 1, cloop, 0, unroll=False)

        for p in range(2):
            @pl.when(nbc > p)
            def _(p=p):
                lb = ((nbc - 1 - p) >> 1) * 2 + p
                c_scatter(lb, p).wait()

    return body


def kernel(input_ids, text_mask, text_table, code_tables):
    ids_t = input_ids.reshape(T, NUM_VQ).T.reshape(-1)
    maski = text_mask.reshape(T).astype(jnp.int32)
    code_flat = code_tables.reshape(NUM_VQ * NUM_AUDIO, D)
    out = _make_kernel()(ids_t, maski, text_table, code_flat)
    return out.reshape(B, S, D)


# speedup vs baseline: 3.8155x; 1.1113x over previous
import functools

import jax
import jax.numpy as jnp
from jax import lax
from jax.experimental import pallas as pl
from jax.experimental.pallas import tpu as pltpu
from jax.experimental.pallas import tpu_sc as plsc

B, S, NUM_VQ = 4, 8192, 4
D = 1024
T = B * S
NUM_AUDIO = 8192
NC, NS = 2, 16
NW = NC * NS
CHUNK = T // NW
LANES = 16
NB = 16
GROWS = NUM_VQ * NB


def _make_kernel():
    mesh = plsc.VectorSubcoreMesh(core_axis_name="c", subcore_axis_name="s")

    @functools.partial(
        pl.kernel,
        out_type=jax.ShapeDtypeStruct((T, D), jnp.float32),
        mesh=mesh,
        scratch_types=[
            pltpu.VMEM((NUM_VQ * CHUNK,), jnp.int32),
            pltpu.VMEM((CHUNK,), jnp.int32),
            pltpu.VMEM((CHUNK,), jnp.int32),
            pltpu.VMEM((CHUNK,), jnp.int32),
            pltpu.VMEM((CHUNK,), jnp.int32),
            pltpu.VMEM((NUM_VQ * CHUNK,), jnp.int32),
            pltpu.VMEM(((NUM_VQ - 1) * NB, D), jnp.float32),
            pltpu.VMEM((NB, D), jnp.float32),
            pltpu.VMEM((NB, D), jnp.float32),
            pltpu.SemaphoreType.DMA,
            pltpu.SemaphoreType.DMA,
            pltpu.SemaphoreType.DMA,
            pltpu.SemaphoreType.DMA,
            pltpu.SemaphoreType.DMA,
            pltpu.SemaphoreType.DMA,
            pltpu.SemaphoreType.DMA,
        ],
    )
    def body(ids_hbm, mask_hbm, text_hbm, code_hbm, out_hbm,
             ids_v, mask_v, tpos, tids, cpos, cids, gbuf, acc0, acc1,
             sga0, sga1, sg1, sg2, sg3, ss0, ss1):
        accs = (acc0, acc1)
        sgas = (sga0, sga1)
        sgs = (None, sg1, sg2, sg3)
        sss = (ss0, ss1)

        wid = lax.axis_index("s") * NC + lax.axis_index("c")
        base = wid * CHUNK
        for i in range(NUM_VQ):
            pltpu.sync_copy(ids_hbm.at[pl.ds(i * T + base, CHUNK)],
                            ids_v.at[pl.ds(i * CHUNK, CHUNK)])
        pltpu.sync_copy(mask_hbm.at[pl.ds(base, CHUNK)], mask_v)

        iota = lax.iota(jnp.int32, LANES)

        def lane_gather(v, idx):
            return lax.gather(
                v, idx[:, None],
                lax.GatherDimensionNumbers(
                    offset_dims=(), collapsed_slice_dims=(0,),
                    start_index_map=(0,)),
                (1,), mode=lax.GatherScatterMode.PROMISE_IN_BOUNDS)

        def prefix_sum(v):
            for d in (1, 2, 4, 8):
                sh = lane_gather(v, jnp.maximum(iota - d, 0))
                v = v + jnp.where(iota >= d, sh, 0)
            return v

        def lower_bound(arr):
            tgt = iota + 1
            lo = jnp.zeros((LANES,), jnp.int32)
            for step in (8, 4, 2, 1):
                v = lane_gather(arr, lo + (step - 1))
                lo = jnp.where(v < tgt, lo + step, lo)
            return lo

        def part(j, carry):
            (bt, ft, cr_tp, cr_ti,
             bc, fc, cr_cp, cr_c0, cr_c1, cr_c2, cr_c3) = carry
            o = j * LANES
            mi = mask_v[pl.ds(o, LANES)]
            pos = base + o + iota
            pt = prefix_sum(mi)
            pc = (1 + iota) - pt
            cnt_t = pt[LANES - 1]
            cnt_c = LANES - cnt_t

            def emit(arr_prefix, cnt, fill, blk, vals, carries, stores):
                src = lower_bound(arr_prefix)
                tot = fill + cnt
                full = (tot >= LANES).astype(jnp.int32)
                new_carries = []
                merged_all = []
                for x, cr in zip(vals, carries):
                    cx = lane_gather(x, src)
                    merged = jnp.where(iota < fill,
                                       cr,
                                       lane_gather(cx, jnp.maximum(iota - fill, 0)))
                    spill = lane_gather(cx, jnp.minimum(iota + LANES - fill,
                                                        LANES - 1))
                    merged_all.append(merged)
                    new_carries.append(jnp.where(full > 0, spill, merged))

                @pl.when(full > 0)
                def _():
                    for st, mv in zip(stores, merged_all):
                        st(blk, mv)
                return blk + full, tot - LANES * full, new_carries

            def t_store_pos(b2, v):
                tpos[pl.ds(b2 * NB, NB)] = v

            def t_store_id(b2, v):
                tids[pl.ds(b2 * NB, NB)] = v

            def c_store_pos(b2, v):
                cpos[pl.ds(b2 * NB, NB)] = v

            def c_store_id(i):
                def st(b2, v):
                    cids[pl.ds(b2 * GROWS + i * NB, NB)] = v
                return st

            id0 = ids_v[pl.ds(o, LANES)]
            bt, ft, (cr_tp, cr_ti) = emit(
                pt, cnt_t, ft, bt, [pos, id0], [cr_tp, cr_ti],
                [t_store_pos, t_store_id])

            cvals = [pos] + [ids_v[pl.ds(i * CHUNK + o, LANES)] + i * NUM_AUDIO
                             for i in range(NUM_VQ)]
            bc, fc, (cr_cp, cr_c0, cr_c1, cr_c2, cr_c3) = emit(
                pc, cnt_c, fc, bc, cvals,
                [cr_cp, cr_c0, cr_c1, cr_c2, cr_c3],
                [c_store_pos] + [c_store_id(i) for i in range(NUM_VQ)])

            return (bt, ft, cr_tp, cr_ti,
                    bc, fc, cr_cp, cr_c0, cr_c1, cr_c2, cr_c3)

        zvec = jnp.zeros((LANES,), jnp.int32)
        z = jnp.int32(0)
        (bt, ft, cr_tp, cr_ti,
         bc, fc, cr_cp, cr_c0, cr_c1, cr_c2, cr_c3) = lax.fori_loop(
            0, CHUNK // LANES, part,
            (z, z, zvec, zvec, z, z, zvec, zvec, zvec, zvec, zvec))

        @pl.when(ft > 0)
        def _flush_text():
            sel = jnp.minimum(iota, ft - 1)
            tpos[pl.ds(bt * NB, NB)] = lane_gather(cr_tp, sel)
            tids[pl.ds(bt * NB, NB)] = lane_gather(cr_ti, sel)

        nbt = bt + (ft > 0).astype(jnp.int32)

        @pl.when(fc > 0)
        def _flush_code():
            sel = jnp.minimum(iota, fc - 1)
            cpos[pl.ds(bc * NB, NB)] = lane_gather(cr_cp, sel)
            for i, cr in enumerate((cr_c0, cr_c1, cr_c2, cr_c3)):
                cids[pl.ds(bc * GROWS + i * NB, NB)] = lane_gather(cr, sel)

        nbc = bc + (fc > 0).astype(jnp.int32)

        def t_gather(b, k):
            return pltpu.make_async_copy(
                text_hbm.at[tids.at[pl.ds(b * NB, NB)]], accs[k], sgas[k])

        def t_scatter(b, k):
            tposv = tpos[pl.ds(b * NB, NB)]
            return pltpu.make_async_copy(accs[k], out_hbm.at[tposv], sss[k])

        for k in range(2):
            @pl.when(k < nbt)
            def _(k=k):
                t_gather(k, k).start()

        def tloop(i, carry):
            for k in range(2):
                b = 2 * i + k

                @pl.when(b < nbt)
                def _(b=b, k=k):
                    t_gather(b, k).wait()
                    t_scatter(b, k).start()
            for k in range(2):
                bn = 2 * i + 2 + k

                @pl.when(bn < nbt)
                def _(bn=bn, k=k, i=i):
                    t_scatter(2 * i + k, k).wait()
                    t_gather(bn, k).start()
            return carry

        lax.fori_loop(0, (nbt + 1) >> 1, tloop, 0, unroll=False)

        for k in range(2):
            @pl.when(nbt > k)
            def _(k=k):
                lb = ((nbt - 1 - k) >> 1) * 2 + k
                t_scatter(lb, k).wait()

        UNROLL = 8

        def c_gather_acc(b, p):
            return pltpu.make_async_copy(
                code_hbm.at[cids.at[pl.ds(b * GROWS, NB)]], accs[p], sgas[p])

        def c_gather_g(b, i):
            return pltpu.make_async_copy(
                code_hbm.at[cids.at[pl.ds(b * GROWS + i * NB, NB)]],
                gbuf.at[pl.ds((i - 1) * NB, NB)], sgs[i])

        def c_scatter(b, p):
            cposv = cpos[pl.ds(b * NB, NB)]
            return pltpu.make_async_copy(accs[p], out_hbm.at[cposv], sss[p])

        def addpass(p, i):
            def ap(c, _):
                for u in range(UNROLL):
                    off = (c * UNROLL + u) * LANES
                    for r in range(NB):
                        plsc.addupdate(
                            accs[p].at[r, pl.ds(off, LANES)],
                            gbuf[(i - 1) * NB + r, pl.ds(off, LANES)])
                return 0
            lax.fori_loop(0, D // LANES // UNROLL, ap, 0, unroll=False)

        @pl.when(nbc > 0)
        def _():
            c_gather_acc(0, 0).start()
            for i in range(1, NUM_VQ):
                c_gather_g(0, i).start()

        def cblock(b, p):
            c_gather_g(b, 1).wait()
            c_gather_acc(b, p).wait()
            addpass(p, 1)

            @pl.when(b + 1 < nbc)
            def _(b=b):
                c_gather_g(b + 1, 1).start()

            @pl.when(b >= 1)
            def _(b=b, p=p):
                c_scatter(b - 1, 1 - p).wait()

            @pl.when(b + 1 < nbc)
            def _(b=b, p=p):
                c_gather_acc(b + 1, 1 - p).start()

            for i in (2, 3):
                c_gather_g(b, i).wait()
                addpass(p, i)

                @pl.when(b + 1 < nbc)
                def _(b=b, i=i):
                    c_gather_g(b + 1, i).start()

            c_scatter(b, p).start()

        def cloop(i2, carry):
            for p in range(2):
                b = 2 * i2 + p

                @pl.when(b < nbc)
                def _(b=b, p=p):
                    cblock(b, p)
            return carry

        lax.fori_loop(0, (nbc + 1) >> 1, cloop, 0, unroll=False)

        for p in range(2):
            @pl.when(jnp.logical_and(nbc > 0, ((nbc - 1) & 1) == p))
            def _(p=p):
                c_scatter(nbc - 1, p).wait()

    return body


def kernel(input_ids, text_mask, text_table, code_tables):
    ids_t = input_ids.reshape(T, NUM_VQ).T.reshape(-1)
    maski = text_mask.reshape(T).astype(jnp.int32)
    code_flat = code_tables.reshape(NUM_VQ * NUM_AUDIO, D)
    out = _make_kernel()(ids_t, maski, text_table, code_flat)
    return out.reshape(B, S, D)


# speedup vs baseline: 4.0038x; 1.0494x over previous
import functools

import jax
import jax.numpy as jnp
from jax import lax
from jax.experimental import pallas as pl
from jax.experimental.pallas import tpu as pltpu
from jax.experimental.pallas import tpu_sc as plsc

B, S, NUM_VQ = 4, 8192, 4
D = 1024
T = B * S
NUM_AUDIO = 8192
NC, NS = 2, 16
NW = NC * NS
CHUNK = T // NW
LANES = 16
NB = 16
GROWS = NUM_VQ * NB


def _make_kernel():
    mesh = plsc.VectorSubcoreMesh(core_axis_name="c", subcore_axis_name="s")

    @functools.partial(
        pl.kernel,
        out_type=jax.ShapeDtypeStruct((T, D), jnp.float32),
        mesh=mesh,
        scratch_types=[
            pltpu.VMEM((NUM_VQ * CHUNK,), jnp.int32),
            pltpu.VMEM((CHUNK,), jnp.int32),
            pltpu.VMEM((CHUNK,), jnp.int32),
            pltpu.VMEM((CHUNK,), jnp.int32),
            pltpu.VMEM((CHUNK,), jnp.int32),
            pltpu.VMEM((NUM_VQ * CHUNK,), jnp.int32),
            pltpu.VMEM(((NUM_VQ - 1) * NB, D), jnp.float32),
            pltpu.VMEM((NB, D), jnp.float32),
            pltpu.VMEM((NB, D), jnp.float32),
            pltpu.SemaphoreType.DMA,
            pltpu.SemaphoreType.DMA,
            pltpu.SemaphoreType.DMA,
            pltpu.SemaphoreType.DMA,
            pltpu.SemaphoreType.DMA,
            pltpu.SemaphoreType.DMA,
            pltpu.SemaphoreType.DMA,
        ],
    )
    def body(ids_hbm, mask_hbm, text_hbm, code_hbm, out_hbm,
             ids_v, mask_v, tpos, tids, cpos, cids, gbuf, acc0, acc1,
             sga0, sga1, sg1, sg2, sg3, ss0, ss1):
        accs = (acc0, acc1)
        sgas = (sga0, sga1)
        sgs = (None, sg1, sg2, sg3)
        sss = (ss0, ss1)

        wid = lax.axis_index("s") * NC + lax.axis_index("c")
        base = wid * CHUNK
        for i in range(NUM_VQ):
            pltpu.sync_copy(ids_hbm.at[pl.ds(i * T + base, CHUNK)],
                            ids_v.at[pl.ds(i * CHUNK, CHUNK)])
        pltpu.sync_copy(mask_hbm.at[pl.ds(base, CHUNK)], mask_v)

        iota = lax.iota(jnp.int32, LANES)

        def lane_gather(v, idx):
            return lax.gather(
                v, idx[:, None],
                lax.GatherDimensionNumbers(
                    offset_dims=(), collapsed_slice_dims=(0,),
                    start_index_map=(0,)),
                (1,), mode=lax.GatherScatterMode.PROMISE_IN_BOUNDS)

        def prefix_sum(v):
            for d in (1, 2, 4, 8):
                sh = lane_gather(v, jnp.maximum(iota - d, 0))
                v = v + jnp.where(iota >= d, sh, 0)
            return v

        def lower_bound(arr):
            tgt = iota + 1
            lo = jnp.zeros((LANES,), jnp.int32)
            for step in (8, 4, 2, 1):
                v = lane_gather(arr, lo + (step - 1))
                lo = jnp.where(v < tgt, lo + step, lo)
            return lo

        def part(j, carry):
            (bt, ft, cr_tp, cr_ti,
             bc, fc, cr_cp, cr_c0, cr_c1, cr_c2, cr_c3) = carry
            o = j * LANES
            mi = mask_v[pl.ds(o, LANES)]
            pos = base + o + iota
            pt = prefix_sum(mi)
            pc = (1 + iota) - pt
            cnt_t = pt[LANES - 1]
            cnt_c = LANES - cnt_t

            def emit(arr_prefix, cnt, fill, blk, vals, carries, stores):
                src = lower_bound(arr_prefix)
                tot = fill + cnt
                full = (tot >= LANES).astype(jnp.int32)
                new_carries = []
                merged_all = []
                for x, cr in zip(vals, carries):
                    cx = lane_gather(x, src)
                    merged = jnp.where(iota < fill,
                                       cr,
                                       lane_gather(cx, jnp.maximum(iota - fill, 0)))
                    spill = lane_gather(cx, jnp.minimum(iota + LANES - fill,
                                                        LANES - 1))
                    merged_all.append(merged)
                    new_carries.append(jnp.where(full > 0, spill, merged))

                @pl.when(full > 0)
                def _():
                    for st, mv in zip(stores, merged_all):
                        st(blk, mv)
                return blk + full, tot - LANES * full, new_carries

            def t_store_pos(b2, v):
                tpos[pl.ds(b2 * NB, NB)] = v

            def t_store_id(b2, v):
                tids[pl.ds(b2 * NB, NB)] = v

            def c_store_pos(b2, v):
                cpos[pl.ds(b2 * NB, NB)] = v

            def c_store_id(i):
                def st(b2, v):
                    cids[pl.ds(b2 * GROWS + i * NB, NB)] = v
                return st

            id0 = ids_v[pl.ds(o, LANES)]
            bt, ft, (cr_tp, cr_ti) = emit(
                pt, cnt_t, ft, bt, [pos, id0], [cr_tp, cr_ti],
                [t_store_pos, t_store_id])

            cvals = [pos] + [ids_v[pl.ds(i * CHUNK + o, LANES)] + i * NUM_AUDIO
                             for i in range(NUM_VQ)]
            bc, fc, (cr_cp, cr_c0, cr_c1, cr_c2, cr_c3) = emit(
                pc, cnt_c, fc, bc, cvals,
                [cr_cp, cr_c0, cr_c1, cr_c2, cr_c3],
                [c_store_pos] + [c_store_id(i) for i in range(NUM_VQ)])

            return (bt, ft, cr_tp, cr_ti,
                    bc, fc, cr_cp, cr_c0, cr_c1, cr_c2, cr_c3)

        zvec = jnp.zeros((LANES,), jnp.int32)
        z = jnp.int32(0)
        (bt, ft, cr_tp, cr_ti,
         bc, fc, cr_cp, cr_c0, cr_c1, cr_c2, cr_c3) = lax.fori_loop(
            0, CHUNK // LANES, part,
            (z, z, zvec, zvec, z, z, zvec, zvec, zvec, zvec, zvec))

        @pl.when(ft > 0)
        def _flush_text():
            sel = jnp.minimum(iota, ft - 1)
            tpos[pl.ds(bt * NB, NB)] = lane_gather(cr_tp, sel)
            tids[pl.ds(bt * NB, NB)] = lane_gather(cr_ti, sel)

        nbt = bt + (ft > 0).astype(jnp.int32)

        @pl.when(fc > 0)
        def _flush_code():
            sel = jnp.minimum(iota, fc - 1)
            cpos[pl.ds(bc * NB, NB)] = lane_gather(cr_cp, sel)
            for i, cr in enumerate((cr_c0, cr_c1, cr_c2, cr_c3)):
                cids[pl.ds(bc * GROWS + i * NB, NB)] = lane_gather(cr, sel)

        nbc = bc + (fc > 0).astype(jnp.int32)

        UNROLL = 8
        nbi = nbc + nbt

        def c_gather_acc(b, p):
            return pltpu.make_async_copy(
                code_hbm.at[cids.at[pl.ds(b * GROWS, NB)]], accs[p], sgas[p])

        def t_gather(tb, p):
            return pltpu.make_async_copy(
                text_hbm.at[tids.at[pl.ds(tb * NB, NB)]], accs[p], sgas[p])

        def c_gather_g(b, i):
            return pltpu.make_async_copy(
                code_hbm.at[cids.at[pl.ds(b * GROWS + i * NB, NB)]],
                gbuf.at[pl.ds((i - 1) * NB, NB)], sgs[i])

        def acc_gather_wait(p):
            pltpu.make_async_copy(
                code_hbm.at[cids.at[pl.ds(0, NB)]], accs[p], sgas[p]).wait()

        def scatter_wait(p):
            pltpu.make_async_copy(accs[p], out_hbm.at[iota], sss[p]).wait()

        def issue_acc_gather(b, p):
            @pl.when(b < nbc)
            def _():
                c_gather_acc(b, p).start()

            @pl.when(jnp.logical_and(b >= nbc, b < nbi))
            def _():
                t_gather(b - nbc, p).start()

        def addpass(p, i):
            def ap(c, _):
                for u in range(UNROLL):
                    off = (c * UNROLL + u) * LANES
                    for r in range(NB):
                        plsc.addupdate(
                            accs[p].at[r, pl.ds(off, LANES)],
                            gbuf[(i - 1) * NB + r, pl.ds(off, LANES)])
                return 0
            lax.fori_loop(0, D // LANES // UNROLL, ap, 0, unroll=False)

        @pl.when(nbi > 0)
        def _():
            issue_acc_gather(0, 0)

        @pl.when(nbc > 0)
        def _():
            for i in range(1, NUM_VQ):
                c_gather_g(0, i).start()

        def block(b, p):
            acc_gather_wait(p)

            @pl.when(b < nbc)
            def _(b=b, p=p):
                c_gather_g(b, 1).wait()
                addpass(p, 1)

                @pl.when(b + 1 < nbc)
                def _():
                    c_gather_g(b + 1, 1).start()

            @pl.when(b >= 1)
            def _(p=p):
                scatter_wait(1 - p)

            issue_acc_gather(b + 1, 1 - p)

            @pl.when(b < nbc)
            def _(b=b, p=p):
                for i in (2, 3):
                    c_gather_g(b, i).wait()
                    addpass(p, i)

                    @pl.when(b + 1 < nbc)
                    def _(i=i):
                        c_gather_g(b + 1, i).start()

                cposv = cpos[pl.ds(b * NB, NB)]
                pltpu.make_async_copy(
                    accs[p], out_hbm.at[cposv], sss[p]).start()

            @pl.when(b >= nbc)
            def _(b=b, p=p):
                tposv = tpos[pl.ds((b - nbc) * NB, NB)]
                pltpu.make_async_copy(
                    accs[p], out_hbm.at[tposv], sss[p]).start()

        def iloop(i2, carry):
            for p in range(2):
                b = 2 * i2 + p

                @pl.when(b < nbi)
                def _(b=b, p=p):
                    block(b, p)
            return carry

        lax.fori_loop(0, (nbi + 1) >> 1, iloop, 0, unroll=False)

        for p in range(2):
            @pl.when(jnp.logical_and(nbi > 0, ((nbi - 1) & 1) == p))
            def _(p=p):
                scatter_wait(p)

    return body


def kernel(input_ids, text_mask, text_table, code_tables):
    ids_t = input_ids.reshape(T, NUM_VQ).T.reshape(-1)
    maski = text_mask.reshape(T).astype(jnp.int32)
    code_flat = code_tables.reshape(NUM_VQ * NUM_AUDIO, D)
    out = _make_kernel()(ids_t, maski, text_table, code_flat)
    return out.reshape(B, S, D)
